# baseline (speedup 1.0000x reference)
import functools

import jax
import jax.numpy as jnp
from jax import lax
from jax.experimental import pallas as pl
from jax.experimental.pallas import tpu as pltpu
from jax.experimental.pallas import tpu_sc as plsc

B = 4096
L = 50
H = 64
NIDX = 2 * B
CW = 8
MB8 = 8
NW = 32
BPW = NIDX // NW
BBLK = 512
N2 = 2 * BBLK
KG = 104


@functools.cache
def _make_sc_gather():
    mesh = plsc.VectorSubcoreMesh(core_axis_name="c", subcore_axis_name="s")

    @functools.partial(
        pl.kernel,
        mesh=mesh,
        compiler_params=pltpu.CompilerParams(
            use_tc_tiling_on_sc=False, needs_layout_passes=False),
        out_type=(
            jax.ShapeDtypeStruct((NIDX, CW), jnp.float32),
            jax.ShapeDtypeStruct((L * MB8, NIDX), jnp.float32),
        ),
        scratch_types=[
            pltpu.VMEM((BPW,), jnp.int32),
            pltpu.VMEM((BPW, CW), jnp.float32),
            pltpu.VMEM((L, BPW), jnp.int32),
            pltpu.VMEM((L * MB8, BPW), jnp.float32),
            pltpu.VMEM((85 * MB8,), jnp.float32),
            pltpu.SemaphoreType.DMA,
            pltpu.SemaphoreType.DMA,
        ],
    )
    def sc_gather(table_hbm, cidx_hbm, base_hbm, seqT_hbm,
                  counts_hbm, emb_hbm,
                  cidx_v, rows_v, seq_v, emb_v, base_v, sem, sem2):
        wid = lax.axis_index("s") * 2 + lax.axis_index("c")
        base = wid * BPW
        pltpu.sync_copy(cidx_hbm.at[pl.ds(base, BPW)], cidx_v)
        seq_cp = pltpu.async_copy(
            seqT_hbm.at[:, pl.ds(base, BPW)], seq_v, sem2)
        pltpu.sync_copy(base_hbm, base_v)
        pltpu.async_copy(table_hbm.at[cidx_v], rows_v, sem).wait()
        pltpu.sync_copy(rows_v, counts_hbm.at[pl.ds(base, BPW)])
        seq_cp.wait()

        zeros16 = jnp.zeros((16,), jnp.float32)
        ones16 = jnp.ones((16,), jnp.float32)

        TUNR = 5

        def step(tt, _):
            for u in range(TUNR):
                t = tt * TUNR + u
                for g in range(BPW // 16):
                    idx16 = seq_v[t, pl.ds(g * 16, 16)]
                    fb = idx16 * MB8
                    for c in range(6):
                        vals = plsc.load_gather(base_v, [fb + c])
                        emb_v[t * MB8 + c, pl.ds(g * 16, 16)] = vals
                    emb_v[t * MB8 + 6, pl.ds(g * 16, 16)] = zeros16
                    emb_v[t * MB8 + 7, pl.ds(g * 16, 16)] = ones16
            return 0

        lax.fori_loop(0, L // TUNR, step, 0)
        pltpu.async_copy(emb_v, emb_hbm.at[:, pl.ds(base, BPW)], sem).wait()

    return sc_gather


def _tc_body(embf_ref, embt_ref, geno_ref, ind_ref, mat_ref, cf_ref, cr_ref,
             Wxh_ref, genopT_ref,
             WgT_ref, WiT_ref, WmT_ref, WfT_ref, WtT_ref,
             WcfT_ref, WcrT_ref, brT_ref, out_ref):
    f32 = jnp.float32
    Mg = jnp.dot(WgT_ref[...], genopT_ref[...],
                 preferred_element_type=f32)
    Wxhv = Wxh_ref[...]

    def zproj(x, h):
        xh = jnp.concatenate([x, h], axis=0)
        return jnp.dot(Wxhv, xh, preferred_element_type=f32)

    def gates(z, c):
        i = 0.5 * jnp.tanh(z[0:H]) + 0.5
        f = 0.5 * jnp.tanh(z[H:2 * H]) + 0.5
        g = jnp.tanh(z[2 * H:3 * H])
        o = 0.5 * jnp.tanh(z[3 * H:4 * H]) + 0.5
        c = f * c + i * g
        h = o * jnp.tanh(c)
        return h, c

    NCH = 4
    CWID = 2 * BBLK // NCH

    def one_step(t, hc):
        hs, cs = hc
        xf = embf_ref[pl.ds(t * MB8, MB8), :]
        xt = embt_ref[pl.ds(t * MB8, MB8), :]
        xparts = [xf[:, 0:CWID], xf[:, CWID:BBLK],
                  xt[:, 0:CWID], xt[:, CWID:BBLK]]
        zs = [zproj(xparts[k], hs[k]) for k in range(NCH)]
        new = [gates(zs[k], cs[k]) for k in range(NCH)]
        return (tuple(n[0] for n in new), tuple(n[1] for n in new))

    UNR = 10

    def step(u, hc):
        for k in range(UNR):
            hc = one_step(u * UNR + k, hc)
        return hc

    z0 = jnp.zeros((H, CWID), f32)
    hs, _ = lax.fori_loop(0, L // UNR, step,
                          ((z0,) * NCH, (z0,) * NCH))
    hf = jnp.concatenate([hs[0], hs[1]], axis=1)
    ht = jnp.concatenate([hs[2], hs[3]], axis=1)
    ohg = (lax.broadcasted_iota(jnp.int32, (KG, BBLK), 0)
           == geno_ref[...]).astype(f32)
    acc = (jnp.dot(Mg, ohg, preferred_element_type=f32)
           + jnp.dot(WfT_ref[...], hf, preferred_element_type=f32)
           + jnp.dot(WtT_ref[...], ht, preferred_element_type=f32)
           + jnp.dot(WcfT_ref[...], cf_ref[...], preferred_element_type=f32)
           + jnp.dot(WcrT_ref[...], cr_ref[...], preferred_element_type=f32)
           + brT_ref[...]
           + jnp.where(ind_ref[...] == 0, WiT_ref[:, 0:1], WiT_ref[:, 1:2])
           + jnp.where(mat_ref[...] == 0, WmT_ref[:, 0:1], WmT_ref[:, 1:2]))
    out_ref[...] = jnp.maximum(acc, 0.0)


def _tc_forward(embT, genoT, indT, matT, cfT, crT,
                Wxh, geno_pT,
                WgT, WiT, WmT, WfT, WtT, WcfT, WcrT, brT):
    nblk = B // BBLK
    col = lambda j: (0, j)
    colr = lambda j: (0, j + nblk)
    full = lambda j: (0, 0)
    in_specs = [
        pl.BlockSpec((L * MB8, BBLK), col),
        pl.BlockSpec((L * MB8, BBLK), colr),
        pl.BlockSpec((1, BBLK), col), pl.BlockSpec((1, BBLK), col),
        pl.BlockSpec((1, BBLK), col),
        pl.BlockSpec((CW, BBLK), col), pl.BlockSpec((CW, BBLK), col),
        pl.BlockSpec((4 * H, MB8 + H), full),
        pl.BlockSpec((4, KG), full),
        pl.BlockSpec((H, 4), full), pl.BlockSpec((H, 2), full),
        pl.BlockSpec((H, 2), full),
        pl.BlockSpec((H, H), full), pl.BlockSpec((H, H), full),
        pl.BlockSpec((H, CW), full), pl.BlockSpec((H, CW), full),
        pl.BlockSpec((H, 1), full),
    ]
    return pl.pallas_call(
        _tc_body,
        grid=(nblk,),
        in_specs=in_specs,
        out_specs=pl.BlockSpec((H, BBLK), col),
        out_shape=jax.ShapeDtypeStruct((H, B), jnp.float32),
    )(embT, embT, genoT, indT, matT, cfT, crT,
      Wxh, geno_pT,
      WgT, WiT, WmT, WfT, WtT, WcfT, WcrT, brT)


def kernel(gobyGenotypeIndex, isIndel, matchesReference, fromSequence,
           toSequence, genotypeCountForwardStrand, genotypeCountReverseStrand,
           geno_table, count_table, base_table, Wx, Wh, b_lstm, W_red, b_red):
    i32 = jnp.int32
    f32 = jnp.float32
    idx_counts = jnp.concatenate(
        [genotypeCountForwardStrand, genotypeCountReverseStrand]).astype(i32)
    tbl8 = jnp.pad(count_table.astype(f32), ((0, 0), (0, CW - 5)))
    bt = base_table.astype(f32)
    base8 = jnp.concatenate(
        [bt, jnp.zeros((85, 1), f32), jnp.ones((85, 1), f32)],
        axis=1).reshape(85 * MB8)
    seqT_cat = jnp.concatenate(
        [fromSequence, toSequence], axis=0).astype(i32).T
    counts8, embT = _make_sc_gather()(tbl8, idx_counts, base8, seqT_cat)
    countsT = counts8.T
    cfT, crT = countsT[:, :B], countsT[:, B:]

    genoT = gobyGenotypeIndex.astype(i32).reshape(1, B)
    indT = isIndel.astype(i32).reshape(1, B)
    matT = matchesReference.astype(i32).reshape(1, B)
    Wx8T = jnp.concatenate(
        [Wx.astype(f32).T, jnp.zeros((4 * H, 1), f32),
         b_lstm.astype(f32).reshape(4 * H, 1)], axis=1)
    WhT = Wh.astype(f32).T
    gate_scale = jnp.concatenate(
        [jnp.full((2 * H, 1), 0.5, f32), jnp.ones((H, 1), f32),
         jnp.full((H, 1), 0.5, f32)], axis=0)
    Wxh = jnp.concatenate([Wx8T, WhT], axis=1) * gate_scale
    geno_pT = jnp.pad(geno_table.astype(f32).T, ((0, 0), (0, KG - 100)))
    Wr = W_red.astype(f32)
    WgT = Wr[0:4].T
    WiT = Wr[4:6].T
    WmT = Wr[6:8].T
    WfT = Wr[8:72].T
    WtT = Wr[72:136].T
    WcfT = jnp.pad(Wr[136:141], ((0, CW - 5), (0, 0))).T
    WcrT = jnp.pad(Wr[141:146], ((0, CW - 5), (0, 0))).T
    brT = b_red.astype(f32).reshape(H, 1)

    outT = _tc_forward(embT, genoT, indT, matT, cfT, crT,
                       Wxh, geno_pT,
                       WgT, WiT, WmT, WfT, WtT, WcfT, WcrT, brT)
    return outT.T


# speedup vs baseline: 1.0304x; 1.0304x over previous
import functools

import jax
import jax.numpy as jnp
from jax import lax
from jax.experimental import pallas as pl
from jax.experimental.pallas import tpu as pltpu
from jax.experimental.pallas import tpu_sc as plsc

B = 4096
L = 50
H = 64
NIDX = 2 * B
CW = 8
MB8 = 8
NW = 32
BPW = NIDX // NW
BBLK = 512
N2 = 2 * BBLK
KG = 104


@functools.cache
def _make_sc_gather():
    mesh = plsc.VectorSubcoreMesh(core_axis_name="c", subcore_axis_name="s")

    @functools.partial(
        pl.kernel,
        mesh=mesh,
        compiler_params=pltpu.CompilerParams(
            use_tc_tiling_on_sc=False, needs_layout_passes=False),
        out_type=(
            jax.ShapeDtypeStruct((NIDX, CW), jnp.float32),
            jax.ShapeDtypeStruct((L * MB8, NIDX), jnp.float32),
        ),
        scratch_types=[
            pltpu.VMEM((BPW,), jnp.int32),
            pltpu.VMEM((BPW, CW), jnp.float32),
            pltpu.VMEM((L, BPW), jnp.int32),
            pltpu.VMEM((L * MB8, BPW), jnp.float32),
            pltpu.VMEM((85 * MB8,), jnp.float32),
            pltpu.SemaphoreType.DMA,
            pltpu.SemaphoreType.DMA,
        ],
    )
    def sc_gather(table_hbm, cidx_hbm, base_hbm, seqT_hbm,
                  counts_hbm, emb_hbm,
                  cidx_v, rows_v, seq_v, emb_v, base_v, sem, sem2):
        wid = lax.axis_index("s") * 2 + lax.axis_index("c")
        base = wid * BPW
        pltpu.sync_copy(cidx_hbm.at[pl.ds(base, BPW)], cidx_v)
        seq_cp = pltpu.async_copy(
            seqT_hbm.at[:, pl.ds(base, BPW)], seq_v, sem2)
        pltpu.sync_copy(base_hbm, base_v)
        pltpu.async_copy(table_hbm.at[cidx_v], rows_v, sem).wait()
        pltpu.sync_copy(rows_v, counts_hbm.at[pl.ds(base, BPW)])
        seq_cp.wait()

        zeros16 = jnp.zeros((16,), jnp.float32)
        ones16 = jnp.ones((16,), jnp.float32)

        def step(t, _):
            for g in range(BPW // 16):
                idx16 = seq_v[t, pl.ds(g * 16, 16)]
                fb = idx16 * MB8
                for c in range(6):
                    vals = plsc.load_gather(base_v, [fb + c])
                    emb_v[t * MB8 + c, pl.ds(g * 16, 16)] = vals
                emb_v[t * MB8 + 6, pl.ds(g * 16, 16)] = zeros16
                emb_v[t * MB8 + 7, pl.ds(g * 16, 16)] = ones16
            return 0

        lax.fori_loop(0, L, step, 0)
        pltpu.async_copy(emb_v, emb_hbm.at[:, pl.ds(base, BPW)], sem).wait()

    return sc_gather


def _tc_body(embf_ref, embt_ref, geno_ref, ind_ref, mat_ref, cf_ref, cr_ref,
             Wxh_ref, genopT_ref,
             WgT_ref, WiT_ref, WmT_ref, WfT_ref, WtT_ref,
             WcfT_ref, WcrT_ref, brT_ref, out_ref):
    f32 = jnp.float32
    Mg = jnp.dot(WgT_ref[...], genopT_ref[...],
                 preferred_element_type=f32)
    Wxhv = Wxh_ref[...]

    def zproj(x, h):
        xh = jnp.concatenate([x, h], axis=0)
        return jnp.dot(Wxhv, xh, preferred_element_type=f32)

    def gates(z, c):
        i = 0.5 * jnp.tanh(z[0:H]) + 0.5
        f = 0.5 * jnp.tanh(z[H:2 * H]) + 0.5
        g = jnp.tanh(z[2 * H:3 * H])
        o = 0.5 * jnp.tanh(z[3 * H:4 * H]) + 0.5
        c = f * c + i * g
        h = o * jnp.tanh(c)
        return h, c

    NCH = 4
    CWID = 2 * BBLK // NCH

    def one_step(t, hc):
        hs, cs = hc
        xf = embf_ref[pl.ds(t * MB8, MB8), :]
        xt = embt_ref[pl.ds(t * MB8, MB8), :]
        xparts = [xf[:, 0:CWID], xf[:, CWID:BBLK],
                  xt[:, 0:CWID], xt[:, CWID:BBLK]]
        zs = [zproj(xparts[k], hs[k]) for k in range(NCH)]
        new = [gates(zs[k], cs[k]) for k in range(NCH)]
        return (tuple(n[0] for n in new), tuple(n[1] for n in new))

    UNR = 25

    def step(u, hc):
        for k in range(UNR):
            hc = one_step(u * UNR + k, hc)
        return hc

    z0 = jnp.zeros((H, CWID), f32)
    hs, _ = lax.fori_loop(0, L // UNR, step,
                          ((z0,) * NCH, (z0,) * NCH))
    hf = jnp.concatenate([hs[0], hs[1]], axis=1)
    ht = jnp.concatenate([hs[2], hs[3]], axis=1)
    ohg = (lax.broadcasted_iota(jnp.int32, (KG, BBLK), 0)
           == geno_ref[...]).astype(f32)
    acc = (jnp.dot(Mg, ohg, preferred_element_type=f32)
           + jnp.dot(WfT_ref[...], hf, preferred_element_type=f32)
           + jnp.dot(WtT_ref[...], ht, preferred_element_type=f32)
           + jnp.dot(WcfT_ref[...], cf_ref[...], preferred_element_type=f32)
           + jnp.dot(WcrT_ref[...], cr_ref[...], preferred_element_type=f32)
           + brT_ref[...]
           + jnp.where(ind_ref[...] == 0, WiT_ref[:, 0:1], WiT_ref[:, 1:2])
           + jnp.where(mat_ref[...] == 0, WmT_ref[:, 0:1], WmT_ref[:, 1:2]))
    out_ref[...] = jnp.maximum(acc, 0.0)


def _tc_forward(embT, genoT, indT, matT, cfT, crT,
                Wxh, geno_pT,
                WgT, WiT, WmT, WfT, WtT, WcfT, WcrT, brT):
    nblk = B // BBLK
    col = lambda j: (0, j)
    colr = lambda j: (0, j + nblk)
    full = lambda j: (0, 0)
    in_specs = [
        pl.BlockSpec((L * MB8, BBLK), col),
        pl.BlockSpec((L * MB8, BBLK), colr),
        pl.BlockSpec((1, BBLK), col), pl.BlockSpec((1, BBLK), col),
        pl.BlockSpec((1, BBLK), col),
        pl.BlockSpec((CW, BBLK), col), pl.BlockSpec((CW, BBLK), col),
        pl.BlockSpec((4 * H, MB8 + H), full),
        pl.BlockSpec((4, KG), full),
        pl.BlockSpec((H, 4), full), pl.BlockSpec((H, 2), full),
        pl.BlockSpec((H, 2), full),
        pl.BlockSpec((H, H), full), pl.BlockSpec((H, H), full),
        pl.BlockSpec((H, CW), full), pl.BlockSpec((H, CW), full),
        pl.BlockSpec((H, 1), full),
    ]
    return pl.pallas_call(
        _tc_body,
        grid=(nblk,),
        in_specs=in_specs,
        out_specs=pl.BlockSpec((H, BBLK), col),
        out_shape=jax.ShapeDtypeStruct((H, B), jnp.float32),
    )(embT, embT, genoT, indT, matT, cfT, crT,
      Wxh, geno_pT,
      WgT, WiT, WmT, WfT, WtT, WcfT, WcrT, brT)


def kernel(gobyGenotypeIndex, isIndel, matchesReference, fromSequence,
           toSequence, genotypeCountForwardStrand, genotypeCountReverseStrand,
           geno_table, count_table, base_table, Wx, Wh, b_lstm, W_red, b_red):
    i32 = jnp.int32
    f32 = jnp.float32
    idx_counts = jnp.concatenate(
        [genotypeCountForwardStrand, genotypeCountReverseStrand]).astype(i32)
    tbl8 = jnp.pad(count_table.astype(f32), ((0, 0), (0, CW - 5)))
    bt = base_table.astype(f32)
    base8 = jnp.concatenate(
        [bt, jnp.zeros((85, 1), f32), jnp.ones((85, 1), f32)],
        axis=1).reshape(85 * MB8)
    seqT_cat = jnp.concatenate(
        [fromSequence, toSequence], axis=0).astype(i32).T
    counts8, embT = _make_sc_gather()(tbl8, idx_counts, base8, seqT_cat)
    countsT = counts8.T
    cfT, crT = countsT[:, :B], countsT[:, B:]

    genoT = gobyGenotypeIndex.astype(i32).reshape(1, B)
    indT = isIndel.astype(i32).reshape(1, B)
    matT = matchesReference.astype(i32).reshape(1, B)
    Wx8T = jnp.concatenate(
        [Wx.astype(f32).T, jnp.zeros((4 * H, 1), f32),
         b_lstm.astype(f32).reshape(4 * H, 1)], axis=1)
    WhT = Wh.astype(f32).T
    gate_scale = jnp.concatenate(
        [jnp.full((2 * H, 1), 0.5, f32), jnp.ones((H, 1), f32),
         jnp.full((H, 1), 0.5, f32)], axis=0)
    Wxh = jnp.concatenate([Wx8T, WhT], axis=1) * gate_scale
    geno_pT = jnp.pad(geno_table.astype(f32).T, ((0, 0), (0, KG - 100)))
    Wr = W_red.astype(f32)
    WgT = Wr[0:4].T
    WiT = Wr[4:6].T
    WmT = Wr[6:8].T
    WfT = Wr[8:72].T
    WtT = Wr[72:136].T
    WcfT = jnp.pad(Wr[136:141], ((0, CW - 5), (0, 0))).T
    WcrT = jnp.pad(Wr[141:146], ((0, CW - 5), (0, 0))).T
    brT = b_red.astype(f32).reshape(H, 1)

    outT = _tc_forward(embT, genoT, indT, matT, cfT, crT,
                       Wxh, geno_pT,
                       WgT, WiT, WmT, WfT, WtT, WcfT, WcrT, brT)
    return outT.T


# speedup vs baseline: 1.0332x; 1.0028x over previous
import functools

import jax
import jax.numpy as jnp
from jax import lax
from jax.experimental import pallas as pl
from jax.experimental.pallas import tpu as pltpu
from jax.experimental.pallas import tpu_sc as plsc

B = 4096
L = 50
H = 64
NIDX = 2 * B
CW = 8
MB8 = 8
NW = 32
BPW = NIDX // NW
BBLK = 512
KG = 104


@functools.cache
def _make_sc_gather():
    mesh = plsc.VectorSubcoreMesh(core_axis_name="c", subcore_axis_name="s")

    @functools.partial(
        pl.kernel,
        mesh=mesh,
        compiler_params=pltpu.CompilerParams(
            use_tc_tiling_on_sc=False, needs_layout_passes=False),
        out_type=(
            jax.ShapeDtypeStruct((NIDX, CW), jnp.float32),
            jax.ShapeDtypeStruct((L * MB8, NIDX), jnp.float32),
        ),
        scratch_types=[
            pltpu.VMEM((BPW,), jnp.int32),
            pltpu.VMEM((BPW, CW), jnp.float32),
            pltpu.VMEM((L, BPW), jnp.int32),
            pltpu.VMEM((L * MB8, BPW), jnp.float32),
            pltpu.VMEM((85 * MB8,), jnp.float32),
            pltpu.SemaphoreType.DMA,
            pltpu.SemaphoreType.DMA,
        ],
    )
    def sc_gather(table_hbm, cidx_hbm, base_hbm, seqT_hbm,
                  counts_hbm, emb_hbm,
                  cidx_v, rows_v, seq_v, emb_v, base_v, sem, sem2):
        wid = lax.axis_index("s") * 2 + lax.axis_index("c")
        base = wid * BPW
        pltpu.sync_copy(cidx_hbm.at[pl.ds(base, BPW)], cidx_v)
        seq_cp = pltpu.async_copy(
            seqT_hbm.at[:, pl.ds(base, BPW)], seq_v, sem2)
        pltpu.sync_copy(base_hbm, base_v)
        pltpu.async_copy(table_hbm.at[cidx_v], rows_v, sem).wait()
        pltpu.sync_copy(rows_v, counts_hbm.at[pl.ds(base, BPW)])
        seq_cp.wait()

        zeros16 = jnp.zeros((16,), jnp.float32)
        ones16 = jnp.ones((16,), jnp.float32)

        def step(t, _):
            for g in range(BPW // 16):
                idx16 = seq_v[t, pl.ds(g * 16, 16)]
                fb = idx16 * MB8
                for c in range(6):
                    vals = plsc.load_gather(base_v, [fb + c])
                    emb_v[t * MB8 + c, pl.ds(g * 16, 16)] = vals
                emb_v[t * MB8 + 6, pl.ds(g * 16, 16)] = zeros16
                emb_v[t * MB8 + 7, pl.ds(g * 16, 16)] = ones16
            return 0

        lax.fori_loop(0, L, step, 0)
        pltpu.async_copy(emb_v, emb_hbm.at[:, pl.ds(base, BPW)], sem).wait()

    return sc_gather


def _tc_body(embf_ref, embt_ref, geno_ref, ind_ref, mat_ref, cf_ref, cr_ref,
             Wxh_ref, genopT_ref,
             WgT_ref, WiT_ref, WmT_ref, WfT_ref, WtT_ref,
             WcfT_ref, WcrT_ref, brT_ref, out_ref):
    f32 = jnp.float32
    Mg = jnp.dot(WgT_ref[...], genopT_ref[...],
                 preferred_element_type=f32)
    Wxhv = Wxh_ref[...]

    def zproj(x, h):
        xh = jnp.concatenate([x, h], axis=0)
        return jnp.dot(Wxhv, xh, preferred_element_type=f32)

    def gates(z, c):
        i = 0.5 * jnp.tanh(z[0:H]) + 0.5
        f = 0.5 * jnp.tanh(z[H:2 * H]) + 0.5
        g = jnp.tanh(z[2 * H:3 * H])
        o = 0.5 * jnp.tanh(z[3 * H:4 * H]) + 0.5
        c = f * c + i * g
        h = o * jnp.tanh(c)
        return h, c

    NCH = 4
    CWID = 2 * BBLK // NCH

    def one_step(t, hc):
        hs, cs = hc
        xf = embf_ref[pl.ds(t * MB8, MB8), :]
        xt = embt_ref[pl.ds(t * MB8, MB8), :]
        xparts = [xf[:, 0:CWID], xf[:, CWID:BBLK],
                  xt[:, 0:CWID], xt[:, CWID:BBLK]]
        zs = [zproj(xparts[k], hs[k]) for k in range(NCH)]
        new = [gates(zs[k], cs[k]) for k in range(NCH)]
        return (tuple(n[0] for n in new), tuple(n[1] for n in new))

    UNR = 25

    def step(u, hc):
        for k in range(UNR):
            hc = one_step(u * UNR + k, hc)
        return hc

    z0 = jnp.zeros((H, CWID), f32)
    hs, _ = lax.fori_loop(0, L // UNR, step,
                          ((z0,) * NCH, (z0,) * NCH))
    hf = jnp.concatenate([hs[0], hs[1]], axis=1)
    ht = jnp.concatenate([hs[2], hs[3]], axis=1)
    ohg = (lax.broadcasted_iota(jnp.int32, (KG, BBLK), 0)
           == geno_ref[...]).astype(f32)
    acc = (jnp.dot(Mg, ohg, preferred_element_type=f32)
           + jnp.dot(WfT_ref[...], hf, preferred_element_type=f32)
           + jnp.dot(WtT_ref[...], ht, preferred_element_type=f32)
           + jnp.dot(WcfT_ref[...], cf_ref[...], preferred_element_type=f32)
           + jnp.dot(WcrT_ref[...], cr_ref[...], preferred_element_type=f32)
           + brT_ref[...]
           + jnp.where(ind_ref[...] == 0, WiT_ref[:, 0:1], WiT_ref[:, 1:2])
           + jnp.where(mat_ref[...] == 0, WmT_ref[:, 0:1], WmT_ref[:, 1:2]))
    out_ref[...] = jnp.maximum(acc, 0.0)


def _tc_forward(embT, genoT, indT, matT, cfT, crT,
                Wxh, geno_pT,
                WgT, WiT, WmT, WfT, WtT, WcfT, WcrT, brT):
    nblk = B // BBLK
    col = lambda j: (0, j)
    colr = lambda j: (0, j + nblk)
    full = lambda j: (0, 0)
    in_specs = [
        pl.BlockSpec((L * MB8, BBLK), col),
        pl.BlockSpec((L * MB8, BBLK), colr),
        pl.BlockSpec((1, BBLK), col), pl.BlockSpec((1, BBLK), col),
        pl.BlockSpec((1, BBLK), col),
        pl.BlockSpec((CW, BBLK), col), pl.BlockSpec((CW, BBLK), col),
        pl.BlockSpec((4 * H, MB8 + H), full),
        pl.BlockSpec((4, KG), full),
        pl.BlockSpec((H, 4), full), pl.BlockSpec((H, 2), full),
        pl.BlockSpec((H, 2), full),
        pl.BlockSpec((H, H), full), pl.BlockSpec((H, H), full),
        pl.BlockSpec((H, CW), full), pl.BlockSpec((H, CW), full),
        pl.BlockSpec((H, 1), full),
    ]
    return pl.pallas_call(
        _tc_body,
        grid=(nblk,),
        in_specs=in_specs,
        out_specs=pl.BlockSpec((H, BBLK), col),
        out_shape=jax.ShapeDtypeStruct((H, B), jnp.float32),
    )(embT, embT, genoT, indT, matT, cfT, crT,
      Wxh, geno_pT,
      WgT, WiT, WmT, WfT, WtT, WcfT, WcrT, brT)


def kernel(gobyGenotypeIndex, isIndel, matchesReference, fromSequence,
           toSequence, genotypeCountForwardStrand, genotypeCountReverseStrand,
           geno_table, count_table, base_table, Wx, Wh, b_lstm, W_red, b_red):
    i32 = jnp.int32
    f32 = jnp.float32
    idx_counts = jnp.concatenate(
        [genotypeCountForwardStrand, genotypeCountReverseStrand]).astype(i32)
    tbl8 = jnp.pad(count_table.astype(f32), ((0, 0), (0, CW - 5)))
    bt = base_table.astype(f32)
    base8 = jnp.concatenate(
        [bt, jnp.zeros((85, 1), f32), jnp.ones((85, 1), f32)],
        axis=1).reshape(85 * MB8)
    seqT_cat = jnp.concatenate(
        [fromSequence, toSequence], axis=0).astype(i32).T
    counts8, embT = _make_sc_gather()(tbl8, idx_counts, base8, seqT_cat)
    countsT = counts8.T
    cfT, crT = countsT[:, :B], countsT[:, B:]

    genoT = gobyGenotypeIndex.astype(i32).reshape(1, B)
    indT = isIndel.astype(i32).reshape(1, B)
    matT = matchesReference.astype(i32).reshape(1, B)
    Wx8T = jnp.concatenate(
        [Wx.astype(f32).T, jnp.zeros((4 * H, 1), f32),
         b_lstm.astype(f32).reshape(4 * H, 1)], axis=1)
    WhT = Wh.astype(f32).T
    gate_scale = jnp.concatenate(
        [jnp.full((2 * H, 1), 0.5, f32), jnp.ones((H, 1), f32),
         jnp.full((H, 1), 0.5, f32)], axis=0)
    Wxh = jnp.concatenate([Wx8T, WhT], axis=1) * gate_scale
    geno_pT = jnp.pad(geno_table.astype(f32).T, ((0, 0), (0, KG - 100)))
    Wr = W_red.astype(f32)
    WgT = Wr[0:4].T
    WiT = Wr[4:6].T
    WmT = Wr[6:8].T
    WfT = Wr[8:72].T
    WtT = Wr[72:136].T
    WcfT = jnp.pad(Wr[136:141], ((0, CW - 5), (0, 0))).T
    WcrT = jnp.pad(Wr[141:146], ((0, CW - 5), (0, 0))).T
    brT = b_red.astype(f32).reshape(H, 1)

    outT = _tc_forward(embT, genoT, indT, matT, cfT, crT,
                       Wxh, geno_pT,
                       WgT, WiT, WmT, WfT, WtT, WcfT, WcrT, brT)
    return outT.T
